# Initial kernel scaffold; baseline (speedup 1.0000x reference)
import jax
import jax.numpy as jnp
from jax.experimental import pallas as pl


def kernel(x, r_peaks, rand_vals):
    raise NotImplementedError("write your pallas kernel here")


# speedup vs baseline: 160.3381x; 160.3381x over previous
import functools

import jax
import jax.numpy as jnp
from jax import lax
from jax.experimental import pallas as pl
from jax.experimental.pallas import tpu as pltpu
from jax.experimental.pallas import tpu_sc as plsc

N = 16_000_000
P = 32_000
D1 = 25
D2 = 166
RATIO = 0.5
NC, NS = 2, 16
NW = NC * NS
RANGE = N // NW
CHUNK = 10_000
NCH = RANGE // CHUNK
GB = 128
BIG = N + 1_000_000


def _sload(ref, i):
    return ref[pl.ds(i, 16)][0]


def _mask_store_runs(buf_v, rel_a, rel_b, fill, ng):
    lane0 = lax.iota(jnp.int32, 16)
    fsplat = jnp.full((16,), fill, dtype=jnp.float32)
    g0 = (rel_a // 16) * 16

    def one(t, _):
        g = g0 + t * 16
        lane = g + lane0
        m = (lane >= rel_a) & (lane < rel_b)
        buf_v[pl.ds(g, 16)] = jnp.where(m, fsplat, buf_v[pl.ds(g, 16)])
        return 0

    lax.fori_loop(0, ng, one, 0)


def _num_groups(rel_a, rel_b, active):
    g0 = (rel_a // 16) * 16
    ng = (rel_b - g0 + 15) // 16
    return jnp.where(active & (rel_b > rel_a), ng, 0)


def _body(x_hbm, r_hbm, rvals_hbm, out_hbm, r_v, rvals_v, fill_v, idx_v,
          buf_v, tail_v, sem):
    wid = lax.axis_index("s") * NC + lax.axis_index("c")
    base = wid * RANGE

    pltpu.sync_copy(r_hbm, r_v.at[pl.ds(0, P)])
    pltpu.sync_copy(rvals_hbm, rvals_v.at[pl.ds(0, P)])

    r_max = _sload(r_v, P - 1)
    m0 = jnp.maximum(r_max + D2 - N, 0)
    wlo = r_max + D2

    def first_gt(val):
        def step(_, lh):
            lo, hi = lh
            mid = jnp.minimum((lo + hi) // 2, P - 1)
            go = lo < hi
            pred = _sload(r_v, mid) > val
            lo2 = jnp.where(go & ~pred, mid + 1, lo)
            hi2 = jnp.where(go & pred, mid, hi)
            return lo2, hi2
        lo, _ = lax.fori_loop(0, 15, step, (jnp.int32(0), jnp.int32(P)))
        return lo

    p_lo = first_gt(base - D2)
    p_hi = first_gt(base + RANGE + D1 - 1)
    wbase = (p_lo // GB) * GB

    nbatch = (p_hi - wbase + GB - 1) // GB

    def gbatch(b, _):
        off = wbase + b * GB
        for g in range(GB // 16):
            rg = r_v[pl.ds(off + g * 16, 16)]
            fi = rg - 26 + jnp.where(rg < 26, N, 0)
            fi = jnp.where(rg == 25, 0, fi)
            idx_v[pl.ds(g * 16, 16)] = jnp.clip(fi, 0, N - 1)
        pltpu.async_copy(x_hbm.at[idx_v], fill_v.at[pl.ds(b * GB, GB)], sem).wait()
        return 0

    lax.fori_loop(0, nbatch, gbatch, 0)

    is_last_worker = wid == NW - 1
    n_early = first_gt(D1 - 1)

    def chunk_body(c, hi_prev):
        cb = base + c * CHUNK
        ce = cb + CHUNK
        pltpu.sync_copy(x_hbm.at[pl.ds(cb, CHUNK)], buf_v)
        tail_v[pl.ds(0, 16)] = buf_v[pl.ds(CHUNK - 32, 16)]
        tail_v[pl.ds(16, 16)] = buf_v[pl.ds(CHUNK - 16, 16)]

        lo_c = jnp.maximum(hi_prev - 1, p_lo)
        hi_c = first_gt(ce + D1 - 1)

        def wbody(pp, _):
            ri = _sload(r_v, pp)
            s = jnp.maximum(ri - D1, m0)
            nxt = jnp.where(pp + 1 < P, _sload(r_v, jnp.minimum(pp + 1, P - 1)), BIG)
            e = jnp.minimum(jnp.minimum(ri + D2, nxt - D1), N)
            a = jnp.maximum(s, cb)
            b = jnp.minimum(e, ce)
            masked = _sload(rvals_v, pp) > RATIO
            ng = _num_groups(a - cb, b - cb, masked)
            _mask_store_runs(buf_v, a - cb, b - cb, _sload(fill_v, pp - wbase), ng)
            return 0

        lax.fori_loop(lo_c, hi_c, wbody, 0)

        run_virt = is_last_worker & (c == NCH - 1)

        def vbody(i, _):
            ri = _sload(r_v, i)
            s = jnp.maximum(N + ri - D1, wlo)
            nxt = _sload(r_v, jnp.minimum(i + 1, P - 1))
            e = jnp.where((i + 1 < n_early), N + nxt - D1, N)
            a = jnp.maximum(s, cb)
            b = jnp.minimum(e, ce)
            masked = _sload(rvals_v, i) > RATIO
            ng = _num_groups(a - cb, b - cb, masked)
            _mask_store_runs(buf_v, a - cb, b - cb, _sload(tail_v, ri + 6), ng)
            return 0

        lax.fori_loop(0, jnp.where(run_virt, n_early, 0), vbody, 0)

        pltpu.sync_copy(buf_v, out_hbm.at[pl.ds(cb, CHUNK)])
        return hi_c

    lax.fori_loop(0, NCH, chunk_body, p_lo)


@jax.jit
def _qtmask_sc(x1d, r_peaks, rand_vals):
    mesh = plsc.VectorSubcoreMesh(core_axis_name="c", subcore_axis_name="s",
                                  num_cores=NC, num_subcores=NS)
    fn = pl.kernel(
        _body,
        out_type=jax.ShapeDtypeStruct((N,), jnp.float32),
        mesh=mesh,
        scratch_types=[
            pltpu.VMEM((P + GB,), jnp.int32),
            pltpu.VMEM((P + 16,), jnp.float32),
            pltpu.VMEM((P + 16,), jnp.float32),
            pltpu.VMEM((GB,), jnp.int32),
            pltpu.VMEM((CHUNK,), jnp.float32),
            pltpu.VMEM((48,), jnp.float32),
            pltpu.SemaphoreType.DMA,
        ],
        name="qtmask_sc",
    )
    return fn(x1d, r_peaks, rand_vals)


def kernel(x, r_peaks, rand_vals):
    x1d = x.reshape(N)
    r32 = r_peaks.astype(jnp.int32)
    out = _qtmask_sc(x1d, r32, rand_vals)
    return out.reshape(1, N)
